# baseline (speedup 1.0000x reference)
import functools

import jax
import jax.numpy as jnp
from jax import lax
from jax.experimental import pallas as pl
from jax.experimental.pallas import tpu as pltpu
from jax.experimental.pallas import tpu_sc as plsc

HIDDEN = 1024
LN_EPS = 1e-12

NC = 2
NS = 16
NW = NC * NS

C = 32
BS = 2048


def _make_gather_kernel(T):
    TW = T // NW
    NCH = TW // C

    mesh = plsc.VectorSubcoreMesh(
        core_axis_name="c", subcore_axis_name="s",
        num_cores=NC, num_subcores=NS)

    NB = 2
    L = 16
    H2 = HIDDEN // 2

    @functools.partial(
        pl.kernel,
        out_type=jax.ShapeDtypeStruct((T, H2), jnp.int32),
        mesh=mesh,
        scratch_types=[
            pltpu.VMEM((TW,), jnp.int32),
            pltpu.VMEM((NB, C, HIDDEN), jnp.float32),
            pltpu.VMEM((NB, C, H2), jnp.int32),
            [pltpu.SemaphoreType.DMA] * NB,
            [pltpu.SemaphoreType.DMA] * NB,
        ],
        compiler_params=pltpu.CompilerParams(needs_layout_passes=False),
    )
    def gather_kernel(ids_hbm, wtab_hbm, out_hbm, idx_v, buf_v, hbuf_v,
                      gsems, ssems):
        wid = lax.axis_index("s") * NC + lax.axis_index("c")
        tok_base = wid * TW
        pltpu.sync_copy(ids_hbm.at[pl.ds(tok_base, TW)], idx_v)
        gathers = [None] * NB
        stores = [None] * NB

        def start_gather(ch):
            b = ch % NB
            gathers[b] = pltpu.async_copy(
                wtab_hbm.at[idx_v.at[pl.ds(ch * C, C)]],
                buf_v.at[b], gsems[b])

        for ch in range(min(NB, NCH)):
            start_gather(ch)
        for ch in range(NCH):
            b = ch % NB
            gathers[b].wait()

            def conv_row(r, carry):
                for j in range(H2 // L):
                    a = buf_v[b, r, pl.ds(L * j, L)]
                    c = buf_v[b, r, pl.ds(H2 + L * j, L)]
                    w16 = plsc.pack(a, c, format=plsc.PackFormat.INTERLEAVED)
                    hbuf_v[b, r, pl.ds(L * j, L)] = plsc.bitcast(
                        w16, jnp.int32)
                return carry

            lax.fori_loop(0, C, conv_row, 0)
            stores[b] = pltpu.async_copy(
                hbuf_v.at[b], out_hbm.at[pl.ds(tok_base + ch * C, C)],
                ssems[b])
            nxt = ch + NB
            if nxt < NCH:
                stores[b].wait()
                start_gather(nxt)
        for ch in range(max(NCH - NB, 0), NCH):
            stores[ch % NB].wait()

    return gather_kernel


def _ln_body(past_ref, x_ref, pos_hbm, g_ref, b_ref, o_ref, pos_v, sem):
    b = pl.program_id(0)
    j = pl.program_id(1)

    @pl.when(jnp.logical_and(b == 0, j == 0))
    def _():
        seq = pos_v.shape[0]
        start = pl.multiple_of(past_ref[0], 8)
        cp = pltpu.make_async_copy(
            pos_hbm.at[pl.ds(start, seq)], pos_v, sem)
        cp.start()
        cp.wait()

    h2 = x_ref.shape[-1]
    d = 2 * h2
    w = x_ref[0]
    xa = lax.bitcast_convert_type(jnp.left_shift(w, 16), jnp.float32)
    xb = lax.bitcast_convert_type(
        jnp.bitwise_and(w, jnp.int32(-65536)), jnp.float32)
    pos = pos_v[pl.ds(j * BS, BS), :]
    xa = xa + pos[:, :h2]
    xb = xb + pos[:, h2:]
    s1 = jnp.sum(xa, axis=-1, keepdims=True) + jnp.sum(
        xb, axis=-1, keepdims=True)
    mean = s1 * (1.0 / d)
    xca = xa - mean
    xcb = xb - mean
    var = (jnp.sum(xca * xca, axis=-1, keepdims=True)
           + jnp.sum(xcb * xcb, axis=-1, keepdims=True)) * (1.0 / d)
    inv = lax.rsqrt(var + LN_EPS)
    o_ref[0, :, pl.ds(0, h2)] = xca * inv * g_ref[pl.ds(0, h2)] + b_ref[
        pl.ds(0, h2)]
    o_ref[0, :, pl.ds(h2, h2)] = xcb * inv * g_ref[pl.ds(h2, h2)] + b_ref[
        pl.ds(h2, h2)]


def _add_pos_layernorm(x, pos_tab, past_arr, gamma, beta):
    batch, seq, h2 = x.shape
    d = 2 * h2
    grid = (batch, seq // BS)
    return pl.pallas_call(
        _ln_body,
        grid_spec=pltpu.PrefetchScalarGridSpec(
            num_scalar_prefetch=1,
            grid=grid,
            in_specs=[
                pl.BlockSpec((1, BS, h2), lambda b, j, p: (b, j, 0)),
                pl.BlockSpec(memory_space=pl.ANY),
                pl.BlockSpec((d,), lambda b, j, p: (0,)),
                pl.BlockSpec((d,), lambda b, j, p: (0,)),
            ],
            out_specs=pl.BlockSpec((1, BS, d), lambda b, j, p: (b, j, 0)),
            scratch_shapes=[
                pltpu.VMEM((seq, d), jnp.float32),
                pltpu.SemaphoreType.DMA,
            ],
        ),
        compiler_params=pltpu.CompilerParams(
            dimension_semantics=("arbitrary", "arbitrary")),
        out_shape=jax.ShapeDtypeStruct((batch, seq, d), jnp.float32),
    )(past_arr, x, pos_tab, gamma, beta)


def kernel(input_ids, past_length, word_embeddings, position_embeddings,
           ln_gamma, ln_beta):
    batch, seq = input_ids.shape
    T = batch * seq
    ids = input_ids.reshape(T).astype(jnp.int32)
    gathered = _make_gather_kernel(T)(ids, word_embeddings)
    past_arr = jnp.asarray(past_length, jnp.int32).reshape(1)
    out = _add_pos_layernorm(
        gathered.reshape(batch, seq, HIDDEN // 2), position_embeddings,
        past_arr, ln_gamma, ln_beta)
    return out


# speedup vs baseline: 1.0455x; 1.0455x over previous
import functools

import jax
import jax.numpy as jnp
from jax import lax
from jax.experimental import pallas as pl
from jax.experimental.pallas import tpu as pltpu
from jax.experimental.pallas import tpu_sc as plsc

HIDDEN = 1024
LN_EPS = 1e-12

NC = 2
NS = 16
NW = NC * NS

C = 32
BS = 2048


def _make_gather_kernel(T):
    TW = T // NW
    NCH = TW // C

    mesh = plsc.VectorSubcoreMesh(
        core_axis_name="c", subcore_axis_name="s",
        num_cores=NC, num_subcores=NS)

    NB = 2
    L = 16
    H2 = HIDDEN // 2

    @functools.partial(
        pl.kernel,
        out_type=jax.ShapeDtypeStruct((T, H2), jnp.int32),
        mesh=mesh,
        scratch_types=[
            pltpu.VMEM((TW,), jnp.int32),
            pltpu.VMEM((NB, C, HIDDEN), jnp.float32),
            pltpu.VMEM((3, C, H2), jnp.int32),
            [pltpu.SemaphoreType.DMA] * NB,
            [pltpu.SemaphoreType.DMA] * 3,
        ],
        compiler_params=pltpu.CompilerParams(needs_layout_passes=False),
    )
    def gather_kernel(ids_hbm, wtab_hbm, out_hbm, idx_v, buf_v, hbuf_v,
                      gsems, ssems):
        wid = lax.axis_index("s") * NC + lax.axis_index("c")
        tok_base = wid * TW
        pltpu.sync_copy(ids_hbm.at[pl.ds(tok_base, TW)], idx_v)
        gathers = [None] * NB
        stores = [None] * 3

        def start_gather(ch):
            b = ch % NB
            gathers[b] = pltpu.async_copy(
                wtab_hbm.at[idx_v.at[pl.ds(ch * C, C)]],
                buf_v.at[b], gsems[b])

        for ch in range(min(NB, NCH)):
            start_gather(ch)
        for ch in range(NCH):
            b = ch % NB
            s = ch % 3
            gathers[b].wait()
            if ch >= 3:
                stores[s].wait()

            def conv_rows(r2, carry):
                for r0 in range(2):
                    r = 2 * r2 + r0
                    for j in range(H2 // L):
                        a = buf_v[b, r, pl.ds(L * j, L)]
                        c = buf_v[b, r, pl.ds(H2 + L * j, L)]
                        w16 = plsc.pack(
                            a, c, format=plsc.PackFormat.INTERLEAVED)
                        hbuf_v[s, r, pl.ds(L * j, L)] = plsc.bitcast(
                            w16, jnp.int32)
                return carry

            lax.fori_loop(0, C // 2, conv_rows, 0)
            nxt = ch + NB
            if nxt < NCH:
                start_gather(nxt)
            stores[s] = pltpu.async_copy(
                hbuf_v.at[s], out_hbm.at[pl.ds(tok_base + ch * C, C)],
                ssems[s])
        for ch in range(max(NCH - 3, 0), NCH):
            stores[ch % 3].wait()

    return gather_kernel


def _ln_body(past_ref, x_ref, pos_hbm, g_ref, b_ref, o_ref, pos_v, sem):
    b = pl.program_id(0)
    j = pl.program_id(1)

    @pl.when(jnp.logical_and(b == 0, j == 0))
    def _():
        seq = pos_v.shape[0]
        start = pl.multiple_of(past_ref[0], 8)
        cp = pltpu.make_async_copy(
            pos_hbm.at[pl.ds(start, seq)], pos_v, sem)
        cp.start()
        cp.wait()

    h2 = x_ref.shape[-1]
    d = 2 * h2
    w = x_ref[0]
    xa = lax.bitcast_convert_type(jnp.left_shift(w, 16), jnp.float32)
    xb = lax.bitcast_convert_type(
        jnp.bitwise_and(w, jnp.int32(-65536)), jnp.float32)
    pos = pos_v[pl.ds(j * BS, BS), :]
    xa = xa + pos[:, :h2]
    xb = xb + pos[:, h2:]
    s1 = jnp.sum(xa, axis=-1, keepdims=True) + jnp.sum(
        xb, axis=-1, keepdims=True)
    mean = s1 * (1.0 / d)
    xca = xa - mean
    xcb = xb - mean
    var = (jnp.sum(xca * xca, axis=-1, keepdims=True)
           + jnp.sum(xcb * xcb, axis=-1, keepdims=True)) * (1.0 / d)
    inv = lax.rsqrt(var + LN_EPS)
    o_ref[0, :, pl.ds(0, h2)] = xca * inv * g_ref[pl.ds(0, h2)] + b_ref[
        pl.ds(0, h2)]
    o_ref[0, :, pl.ds(h2, h2)] = xcb * inv * g_ref[pl.ds(h2, h2)] + b_ref[
        pl.ds(h2, h2)]


def _add_pos_layernorm(x, pos_tab, past_arr, gamma, beta):
    batch, seq, h2 = x.shape
    d = 2 * h2
    grid = (batch, seq // BS)
    return pl.pallas_call(
        _ln_body,
        grid_spec=pltpu.PrefetchScalarGridSpec(
            num_scalar_prefetch=1,
            grid=grid,
            in_specs=[
                pl.BlockSpec((1, BS, h2), lambda b, j, p: (b, j, 0)),
                pl.BlockSpec(memory_space=pl.ANY),
                pl.BlockSpec((d,), lambda b, j, p: (0,)),
                pl.BlockSpec((d,), lambda b, j, p: (0,)),
            ],
            out_specs=pl.BlockSpec((1, BS, d), lambda b, j, p: (b, j, 0)),
            scratch_shapes=[
                pltpu.VMEM((seq, d), jnp.float32),
                pltpu.SemaphoreType.DMA,
            ],
        ),
        compiler_params=pltpu.CompilerParams(
            dimension_semantics=("arbitrary", "arbitrary")),
        out_shape=jax.ShapeDtypeStruct((batch, seq, d), jnp.float32),
    )(past_arr, x, pos_tab, gamma, beta)


def kernel(input_ids, past_length, word_embeddings, position_embeddings,
           ln_gamma, ln_beta):
    batch, seq = input_ids.shape
    T = batch * seq
    ids = input_ids.reshape(T).astype(jnp.int32)
    gathered = _make_gather_kernel(T)(ids, word_embeddings)
    past_arr = jnp.asarray(past_length, jnp.int32).reshape(1)
    out = _add_pos_layernorm(
        gathered.reshape(batch, seq, HIDDEN // 2), position_embeddings,
        past_arr, ln_gamma, ln_beta)
    return out


# speedup vs baseline: 1.3282x; 1.2704x over previous
import functools

import jax
import jax.numpy as jnp
from jax import lax
from jax.experimental import pallas as pl
from jax.experimental.pallas import tpu as pltpu
from jax.experimental.pallas import tpu_sc as plsc

HIDDEN = 1024
LN_EPS = 1e-12

NC = 2
NS = 16
NW = NC * NS

C = 32
BS = 2048


def _make_gather_kernel(T):
    TW = T // NW
    NCH = TW // C

    mesh = plsc.VectorSubcoreMesh(
        core_axis_name="c", subcore_axis_name="s",
        num_cores=NC, num_subcores=NS)

    NB = 3

    @functools.partial(
        pl.kernel,
        out_type=jax.ShapeDtypeStruct((T, HIDDEN), jnp.float32),
        mesh=mesh,
        scratch_types=[
            pltpu.VMEM((TW,), jnp.int32),
            pltpu.VMEM((NB, C, HIDDEN), jnp.float32),
            [pltpu.SemaphoreType.DMA] * NB,
            [pltpu.SemaphoreType.DMA] * NB,
        ],
        compiler_params=pltpu.CompilerParams(needs_layout_passes=False),
    )
    def gather_kernel(ids_hbm, wtab_hbm, out_hbm, idx_v, buf_v, gsems, ssems):
        wid = lax.axis_index("s") * NC + lax.axis_index("c")
        tok_base = wid * TW
        pltpu.sync_copy(ids_hbm.at[pl.ds(tok_base, TW)], idx_v)
        gathers = [None] * NB
        stores = [None] * NB

        def start_gather(ch):
            b = ch % NB
            gathers[b] = pltpu.async_copy(
                wtab_hbm.at[idx_v.at[pl.ds(ch * C, C)]],
                buf_v.at[b], gsems[b])

        for ch in range(min(NB, NCH)):
            start_gather(ch)
        for ch in range(NCH):
            b = ch % NB
            gathers[b].wait()
            stores[b] = pltpu.async_copy(
                buf_v.at[b], out_hbm.at[pl.ds(tok_base + ch * C, C)],
                ssems[b])
            nxt = ch + NB
            if nxt < NCH:
                stores[b].wait()
                start_gather(nxt)
        for ch in range(max(NCH - NB, 0), NCH):
            stores[ch % NB].wait()

    return gather_kernel


def _ln_body(past_ref, x_ref, pos_hbm, g_ref, b_ref, o_ref, pos_v, sem):
    b = pl.program_id(0)
    j = pl.program_id(1)

    @pl.when(jnp.logical_and(b == 0, j == 0))
    def _():
        seq = pos_v.shape[0]
        start = pl.multiple_of(past_ref[0], 8)
        cp = pltpu.make_async_copy(
            pos_hbm.at[pl.ds(start, seq)], pos_v, sem)
        cp.start()
        cp.wait()

    x = x_ref[0] + pos_v[pl.ds(j * BS, BS), :]
    mean = jnp.mean(x, axis=-1, keepdims=True)
    xc = x - mean
    var = jnp.mean(xc * xc, axis=-1, keepdims=True)
    y = xc * lax.rsqrt(var + LN_EPS)
    o_ref[0] = y * g_ref[...] + b_ref[...]


def _add_pos_layernorm(x, pos_tab, past_arr, gamma, beta):
    batch, seq, d = x.shape
    grid = (batch, seq // BS)
    return pl.pallas_call(
        _ln_body,
        grid_spec=pltpu.PrefetchScalarGridSpec(
            num_scalar_prefetch=1,
            grid=grid,
            in_specs=[
                pl.BlockSpec((1, BS, d), lambda b, j, p: (b, j, 0)),
                pl.BlockSpec(memory_space=pl.ANY),
                pl.BlockSpec((d,), lambda b, j, p: (0,)),
                pl.BlockSpec((d,), lambda b, j, p: (0,)),
            ],
            out_specs=pl.BlockSpec((1, BS, d), lambda b, j, p: (b, j, 0)),
            scratch_shapes=[
                pltpu.VMEM((seq, d), jnp.float32),
                pltpu.SemaphoreType.DMA,
            ],
        ),
        compiler_params=pltpu.CompilerParams(
            dimension_semantics=("arbitrary", "arbitrary")),
        out_shape=jax.ShapeDtypeStruct((batch, seq, d), jnp.float32),
    )(past_arr, x, pos_tab, gamma, beta)


def kernel(input_ids, past_length, word_embeddings, position_embeddings,
           ln_gamma, ln_beta):
    batch, seq = input_ids.shape
    T = batch * seq
    ids = input_ids.reshape(T).astype(jnp.int32)
    gathered = _make_gather_kernel(T)(ids, word_embeddings)
    past_arr = jnp.asarray(past_length, jnp.int32).reshape(1)
    out = _add_pos_layernorm(
        gathered.reshape(batch, seq, HIDDEN), position_embeddings,
        past_arr, ln_gamma, ln_beta)
    return out


# speedup vs baseline: 1.3325x; 1.0032x over previous
import functools

import jax
import jax.numpy as jnp
from jax import lax
from jax.experimental import pallas as pl
from jax.experimental.pallas import tpu as pltpu
from jax.experimental.pallas import tpu_sc as plsc

HIDDEN = 1024
LN_EPS = 1e-12

NC = 2
NS = 16
NW = NC * NS

C = 32
BS = 1024


def _make_gather_kernel(T):
    TW = T // NW
    NCH = TW // C

    mesh = plsc.VectorSubcoreMesh(
        core_axis_name="c", subcore_axis_name="s",
        num_cores=NC, num_subcores=NS)

    NB = 3

    @functools.partial(
        pl.kernel,
        out_type=jax.ShapeDtypeStruct((T, HIDDEN), jnp.float32),
        mesh=mesh,
        scratch_types=[
            pltpu.VMEM((TW,), jnp.int32),
            pltpu.VMEM((NB, C, HIDDEN), jnp.float32),
            [pltpu.SemaphoreType.DMA] * NB,
            [pltpu.SemaphoreType.DMA] * NB,
        ],
        compiler_params=pltpu.CompilerParams(needs_layout_passes=False),
    )
    def gather_kernel(ids_hbm, wtab_hbm, out_hbm, idx_v, buf_v, gsems, ssems):
        wid = lax.axis_index("s") * NC + lax.axis_index("c")
        tok_base = wid * TW
        pltpu.sync_copy(ids_hbm.at[pl.ds(tok_base, TW)], idx_v)
        gathers = [None] * NB
        stores = [None] * NB

        def start_gather(ch):
            b = ch % NB
            gathers[b] = pltpu.async_copy(
                wtab_hbm.at[idx_v.at[pl.ds(ch * C, C)]],
                buf_v.at[b], gsems[b])

        for ch in range(min(NB, NCH)):
            start_gather(ch)
        for ch in range(NCH):
            b = ch % NB
            gathers[b].wait()
            stores[b] = pltpu.async_copy(
                buf_v.at[b], out_hbm.at[pl.ds(tok_base + ch * C, C)],
                ssems[b])
            nxt = ch + NB
            if nxt < NCH:
                stores[b].wait()
                start_gather(nxt)
        for ch in range(max(NCH - NB, 0), NCH):
            stores[ch % NB].wait()

    return gather_kernel


def _ln_body(past_ref, x_ref, pos_hbm, g_ref, b_ref, o_ref, pos_v, sem):
    b = pl.program_id(0)
    j = pl.program_id(1)

    @pl.when(jnp.logical_and(b == 0, j == 0))
    def _():
        seq = pos_v.shape[0]
        start = pl.multiple_of(past_ref[0], 8)
        cp = pltpu.make_async_copy(
            pos_hbm.at[pl.ds(start, seq)], pos_v, sem)
        cp.start()
        cp.wait()

    x = x_ref[0] + pos_v[pl.ds(j * BS, BS), :]
    mean = jnp.mean(x, axis=-1, keepdims=True)
    xc = x - mean
    var = jnp.mean(xc * xc, axis=-1, keepdims=True)
    y = xc * lax.rsqrt(var + LN_EPS)
    o_ref[0] = y * g_ref[...] + b_ref[...]


def _add_pos_layernorm(x, pos_tab, past_arr, gamma, beta):
    batch, seq, d = x.shape
    grid = (batch, seq // BS)
    return pl.pallas_call(
        _ln_body,
        grid_spec=pltpu.PrefetchScalarGridSpec(
            num_scalar_prefetch=1,
            grid=grid,
            in_specs=[
                pl.BlockSpec((1, BS, d), lambda b, j, p: (b, j, 0)),
                pl.BlockSpec(memory_space=pl.ANY),
                pl.BlockSpec((d,), lambda b, j, p: (0,)),
                pl.BlockSpec((d,), lambda b, j, p: (0,)),
            ],
            out_specs=pl.BlockSpec((1, BS, d), lambda b, j, p: (b, j, 0)),
            scratch_shapes=[
                pltpu.VMEM((seq, d), jnp.float32),
                pltpu.SemaphoreType.DMA,
            ],
        ),
        compiler_params=pltpu.CompilerParams(
            dimension_semantics=("arbitrary", "arbitrary")),
        out_shape=jax.ShapeDtypeStruct((batch, seq, d), jnp.float32),
    )(past_arr, x, pos_tab, gamma, beta)


def kernel(input_ids, past_length, word_embeddings, position_embeddings,
           ln_gamma, ln_beta):
    batch, seq = input_ids.shape
    T = batch * seq
    ids = input_ids.reshape(T).astype(jnp.int32)
    gathered = _make_gather_kernel(T)(ids, word_embeddings)
    past_arr = jnp.asarray(past_length, jnp.int32).reshape(1)
    out = _add_pos_layernorm(
        gathered.reshape(batch, seq, HIDDEN), position_embeddings,
        past_arr, ln_gamma, ln_beta)
    return out
